# baseline (speedup 1.0000x reference)
import functools

import jax
import jax.numpy as jnp
from jax import lax
from jax.experimental import pallas as pl
from jax.experimental.pallas import tpu as pltpu
from jax.experimental.pallas import tpu_sc as plsc

B = 16
C = 5
PIX = 512 * 512
NW = 32
CHUNK = PIX // NW
L = 16
VECS = CHUNK // L

_mesh = plsc.VectorSubcoreMesh(core_axis_name="c", subcore_axis_name="s")


def _sc_body(pred_hbm, heart_hbm, out_hbm, h_v, p1_v, p2_v, p3_v, p4_v, out_v):
    cid = lax.axis_index("c")
    sid = lax.axis_index("s")
    wid = sid * 2 + cid
    base_w = wid * CHUNK

    zero = jnp.zeros((L,), jnp.float32)
    one = jnp.ones((L,), jnp.float32)

    def batch_body(b, accs):
        hb = pl.multiple_of(b * PIX + base_w, 8)
        pltpu.sync_copy(heart_hbm.at[pl.ds(hb, CHUNK)], h_v)
        pb = b * C * PIX + base_w
        pltpu.sync_copy(pred_hbm.at[pl.ds(pl.multiple_of(pb + 1 * PIX, 8), CHUNK)], p1_v)
        pltpu.sync_copy(pred_hbm.at[pl.ds(pl.multiple_of(pb + 2 * PIX, 8), CHUNK)], p2_v)
        pltpu.sync_copy(pred_hbm.at[pl.ds(pl.multiple_of(pb + 3 * PIX, 8), CHUNK)], p3_v)
        pltpu.sync_copy(pred_hbm.at[pl.ds(pl.multiple_of(pb + 4 * PIX, 8), CHUNK)], p4_v)

        def vec_body(i, acc2):
            a0, a1, a2, a3, a4 = acc2
            s = pl.ds(i * L, L)
            m = h_v[s] == 1
            a0 = a0 + jnp.where(m, p1_v[s], zero)
            a1 = a1 + jnp.where(m, p2_v[s], zero)
            a2 = a2 + jnp.where(m, p3_v[s], zero)
            a3 = a3 + jnp.where(m, p4_v[s], zero)
            a4 = a4 + jnp.where(m, one, zero)
            return (a0, a1, a2, a3, a4)

        return lax.fori_loop(0, VECS, vec_body, accs)

    a0, a1, a2, a3, a4 = lax.fori_loop(0, B, batch_body, (zero,) * 5)
    out_v[0, :] = a0
    out_v[1, :] = a1
    out_v[2, :] = a2
    out_v[3, :] = a3
    out_v[4, :] = a4
    pltpu.sync_copy(out_v, out_hbm.at[wid])


_sc_call = functools.partial(
    pl.kernel,
    out_type=jax.ShapeDtypeStruct((NW, 5, L), jnp.float32),
    mesh=_mesh,
    scratch_types=[
        pltpu.VMEM((CHUNK,), jnp.int32),
        pltpu.VMEM((CHUNK,), jnp.float32),
        pltpu.VMEM((CHUNK,), jnp.float32),
        pltpu.VMEM((CHUNK,), jnp.float32),
        pltpu.VMEM((CHUNK,), jnp.float32),
        pltpu.VMEM((5, L), jnp.float32),
    ],
)(_sc_body)


@jax.jit
def kernel(predictions, heart):
    pred1d = predictions.reshape(-1)
    heart1d = heart.reshape(-1)
    partials = _sc_call(pred1d, heart1d)
    sums = jnp.sum(partials, axis=(0, 2))
    n = sums[4]
    probs = sums[:4] / n
    return jnp.mean((probs - jnp.float32(0.25)) ** 2)


# speedup vs baseline: 1.4990x; 1.4990x over previous
import functools

import jax
import jax.numpy as jnp
from jax import lax
from jax.experimental import pallas as pl
from jax.experimental.pallas import tpu as pltpu
from jax.experimental.pallas import tpu_sc as plsc

B = 16
C = 5
PIX = 512 * 512
NW = 32
CHUNK = PIX // NW
L = 16
U = 4
VECS = CHUNK // L

_mesh = plsc.VectorSubcoreMesh(core_axis_name="c", subcore_axis_name="s")


def _sc_body(pred_hbm, heart_hbm, out_hbm,
             h0, p10, p20, p30, p40,
             h1, p11, p21, p31, p41,
             out_v, sem_a, sem_b):
    bufs = ((h0, p10, p20, p30, p40, sem_a),
            (h1, p11, p21, p31, p41, sem_b))
    wid = lax.axis_index("s") * 2 + lax.axis_index("c")
    base_w = wid * CHUNK

    zero = jnp.zeros((L,), jnp.float32)
    one = jnp.ones((L,), jnp.float32)

    def issue(b, slot):
        h_v, p1_v, p2_v, p3_v, p4_v, sem = bufs[slot]
        hb = pl.multiple_of(b * PIX + base_w, 8)
        cps = [pltpu.async_copy(heart_hbm.at[pl.ds(hb, CHUNK)], h_v, sem)]
        pb = b * C * PIX + base_w
        for c, dst in ((1, p1_v), (2, p2_v), (3, p3_v), (4, p4_v)):
            src = pred_hbm.at[pl.ds(pl.multiple_of(pb + c * PIX, 8), CHUNK)]
            cps.append(pltpu.async_copy(src, dst, sem))
        return cps

    def compute(slot, accs):
        h_v, p1_v, p2_v, p3_v, p4_v, _ = bufs[slot]

        def vec_body(i, acc2):
            a0, a1, a2, a3, a4 = acc2
            base = i * (L * U)
            for u in range(U):
                s = pl.ds(base + u * L, L)
                m = h_v[s] == 1
                a0 = a0 + jnp.where(m, p1_v[s], zero)
                a1 = a1 + jnp.where(m, p2_v[s], zero)
                a2 = a2 + jnp.where(m, p3_v[s], zero)
                a3 = a3 + jnp.where(m, p4_v[s], zero)
                a4 = a4 + jnp.where(m, one, zero)
            return (a0, a1, a2, a3, a4)

        return lax.fori_loop(0, VECS // U, vec_body, accs)

    accs = (zero,) * 5
    cps = issue(0, 0)
    for b in range(B):
        if b + 1 < B:
            nxt = issue(b + 1, (b + 1) % 2)
        for cp in cps:
            cp.wait()
        accs = compute(b % 2, accs)
        if b + 1 < B:
            cps = nxt

    a0, a1, a2, a3, a4 = accs
    out_v[0, :] = a0
    out_v[1, :] = a1
    out_v[2, :] = a2
    out_v[3, :] = a3
    out_v[4, :] = a4
    pltpu.sync_copy(out_v, out_hbm.at[wid])


_sc_call = functools.partial(
    pl.kernel,
    out_type=jax.ShapeDtypeStruct((NW, 5, L), jnp.float32),
    mesh=_mesh,
    scratch_types=[
        pltpu.VMEM((CHUNK,), jnp.int32),
        pltpu.VMEM((CHUNK,), jnp.float32),
        pltpu.VMEM((CHUNK,), jnp.float32),
        pltpu.VMEM((CHUNK,), jnp.float32),
        pltpu.VMEM((CHUNK,), jnp.float32),
        pltpu.VMEM((CHUNK,), jnp.int32),
        pltpu.VMEM((CHUNK,), jnp.float32),
        pltpu.VMEM((CHUNK,), jnp.float32),
        pltpu.VMEM((CHUNK,), jnp.float32),
        pltpu.VMEM((CHUNK,), jnp.float32),
        pltpu.VMEM((5, L), jnp.float32),
        pltpu.SemaphoreType.DMA,
        pltpu.SemaphoreType.DMA,
    ],
)(_sc_body)


@jax.jit
def kernel(predictions, heart):
    pred1d = predictions.reshape(-1)
    heart1d = heart.reshape(-1)
    partials = _sc_call(pred1d, heart1d)
    sums = jnp.sum(partials, axis=(0, 2))
    n = sums[4]
    probs = sums[:4] / n
    return jnp.mean((probs - jnp.float32(0.25)) ** 2)


# speedup vs baseline: 3.3828x; 2.2567x over previous
import functools

import jax
import jax.numpy as jnp
from jax import lax
from jax.experimental import pallas as pl
from jax.experimental.pallas import tpu as pltpu
from jax.experimental.pallas import tpu_sc as plsc

B = 16
C = 5
H = 512
W = 512
NW = 32
ROWS = H // NW
L = 16
U = 4
JW = W // (L * U)

_mesh = plsc.VectorSubcoreMesh(core_axis_name="c", subcore_axis_name="s")


def _sc_body(pred_hbm, heart_hbm, out_hbm,
             h0, p0, h1, p1,
             out_v, sem_a, sem_b):
    bufs = ((h0, p0, sem_a), (h1, p1, sem_b))
    wid = lax.axis_index("s") * 2 + lax.axis_index("c")
    row0 = wid * ROWS

    zero = jnp.zeros((L,), jnp.float32)
    one = jnp.ones((L,), jnp.float32)

    def issue(b, slot):
        h_v, p_v, sem = bufs[slot]
        cps = [
            pltpu.async_copy(heart_hbm.at[b, pl.ds(row0, ROWS), :], h_v, sem),
            pltpu.async_copy(
                pred_hbm.at[b, pl.ds(1, 4), pl.ds(row0, ROWS), :], p_v, sem),
        ]
        return cps

    def compute(slot, accs):
        h_v, p_v, _ = bufs[slot]

        def row_body(r, acc_r):
            def vec_body(j, acc2):
                a0, a1, a2, a3, a4 = acc2
                base = j * (L * U)
                for u in range(U):
                    s = pl.ds(base + u * L, L)
                    m = h_v[r, s] == 1
                    a0 = a0 + jnp.where(m, p_v[0, r, s], zero)
                    a1 = a1 + jnp.where(m, p_v[1, r, s], zero)
                    a2 = a2 + jnp.where(m, p_v[2, r, s], zero)
                    a3 = a3 + jnp.where(m, p_v[3, r, s], zero)
                    a4 = a4 + jnp.where(m, one, zero)
                return (a0, a1, a2, a3, a4)

            return lax.fori_loop(0, JW, vec_body, acc_r)

        return lax.fori_loop(0, ROWS, row_body, accs)

    accs = (zero,) * 5
    cps = issue(0, 0)
    for b in range(B):
        if b + 1 < B:
            nxt = issue(b + 1, (b + 1) % 2)
        for cp in cps:
            cp.wait()
        accs = compute(b % 2, accs)
        if b + 1 < B:
            cps = nxt

    a0, a1, a2, a3, a4 = accs
    out_v[0, :] = a0
    out_v[1, :] = a1
    out_v[2, :] = a2
    out_v[3, :] = a3
    out_v[4, :] = a4
    pltpu.sync_copy(out_v, out_hbm.at[wid])


_sc_call = functools.partial(
    pl.kernel,
    out_type=jax.ShapeDtypeStruct((NW, 5, L), jnp.float32),
    mesh=_mesh,
    scratch_types=[
        pltpu.VMEM((ROWS, W), jnp.int32),
        pltpu.VMEM((4, ROWS, W), jnp.float32),
        pltpu.VMEM((ROWS, W), jnp.int32),
        pltpu.VMEM((4, ROWS, W), jnp.float32),
        pltpu.VMEM((5, L), jnp.float32),
        pltpu.SemaphoreType.DMA,
        pltpu.SemaphoreType.DMA,
    ],
)(_sc_body)


@jax.jit
def kernel(predictions, heart):
    heart3d = jnp.squeeze(heart, axis=1)
    partials = _sc_call(predictions, heart3d)
    sums = jnp.sum(partials, axis=(0, 2))
    n = sums[4]
    probs = sums[:4] / n
    return jnp.mean((probs - jnp.float32(0.25)) ** 2)


# speedup vs baseline: 4.2409x; 1.2537x over previous
import functools

import jax
import jax.numpy as jnp
from jax import lax
from jax.experimental import pallas as pl
from jax.experimental.pallas import tpu as pltpu
from jax.experimental.pallas import tpu_sc as plsc

B = 16
C = 5
H = 512
W = 512
BT = 12
BS = B - BT
NW = 32
ROWS = H // NW
L = 16
U = 4
JW = W // (L * U)

_mesh = plsc.VectorSubcoreMesh(core_axis_name="c", subcore_axis_name="s")


def _sc_body(pred_hbm, heart_hbm, out_hbm,
             h0, p0, h1, p1,
             out_v, sem_a, sem_b):
    bufs = ((h0, p0, sem_a), (h1, p1, sem_b))
    wid = lax.axis_index("s") * 2 + lax.axis_index("c")
    row0 = wid * ROWS

    zero = jnp.zeros((L,), jnp.float32)
    one = jnp.ones((L,), jnp.float32)

    def issue(b, slot):
        h_v, p_v, sem = bufs[slot]
        return [
            pltpu.async_copy(heart_hbm.at[b, pl.ds(row0, ROWS), :], h_v, sem),
            pltpu.async_copy(
                pred_hbm.at[b, pl.ds(1, 4), pl.ds(row0, ROWS), :], p_v, sem),
        ]

    def compute(slot, accs):
        h_v, p_v, _ = bufs[slot]

        def row_body(r, acc_r):
            def vec_body(j, acc2):
                a0, a1, a2, a3, a4 = acc2
                base = j * (L * U)
                for u in range(U):
                    s = pl.ds(base + u * L, L)
                    m = h_v[r, s] == 1
                    a0 = a0 + jnp.where(m, p_v[0, r, s], zero)
                    a1 = a1 + jnp.where(m, p_v[1, r, s], zero)
                    a2 = a2 + jnp.where(m, p_v[2, r, s], zero)
                    a3 = a3 + jnp.where(m, p_v[3, r, s], zero)
                    a4 = a4 + jnp.where(m, one, zero)
                return (a0, a1, a2, a3, a4)

            return lax.fori_loop(0, JW, vec_body, acc_r)

        return lax.fori_loop(0, ROWS, row_body, accs)

    accs = (zero,) * 5
    cps = issue(BT, 0)
    for i, b in enumerate(range(BT, B)):
        if b + 1 < B:
            nxt = issue(b + 1, (i + 1) % 2)
        for cp in cps:
            cp.wait()
        accs = compute(i % 2, accs)
        if b + 1 < B:
            cps = nxt

    a0, a1, a2, a3, a4 = accs
    out_v[0, :] = a0
    out_v[1, :] = a1
    out_v[2, :] = a2
    out_v[3, :] = a3
    out_v[4, :] = a4
    pltpu.sync_copy(out_v, out_hbm.at[wid])


_sc_call = functools.partial(
    pl.kernel,
    out_type=jax.ShapeDtypeStruct((NW, 5, L), jnp.float32),
    mesh=_mesh,
    scratch_types=[
        pltpu.VMEM((ROWS, W), jnp.int32),
        pltpu.VMEM((4, ROWS, W), jnp.float32),
        pltpu.VMEM((ROWS, W), jnp.int32),
        pltpu.VMEM((4, ROWS, W), jnp.float32),
        pltpu.VMEM((5, L), jnp.float32),
        pltpu.SemaphoreType.DMA,
        pltpu.SemaphoreType.DMA,
    ],
)(_sc_body)


def _tc_body(h_ref, p1_ref, p2_ref, p3_ref, p4_ref, out_ref):
    mf = (h_ref[0, 0] == 1).astype(jnp.float32)
    out_ref[0, 0, 0] = jnp.sum(p1_ref[0, 0] * mf)
    out_ref[0, 0, 1] = jnp.sum(p2_ref[0, 0] * mf)
    out_ref[0, 0, 2] = jnp.sum(p3_ref[0, 0] * mf)
    out_ref[0, 0, 3] = jnp.sum(p4_ref[0, 0] * mf)
    out_ref[0, 0, 4] = jnp.sum(mf)


_tc_call = pl.pallas_call(
    _tc_body,
    grid=(BT,),
    in_specs=[
        pl.BlockSpec((1, 1, H, W), lambda b: (b, 0, 0, 0)),
        pl.BlockSpec((1, 1, H, W), lambda b: (b, 1, 0, 0)),
        pl.BlockSpec((1, 1, H, W), lambda b: (b, 2, 0, 0)),
        pl.BlockSpec((1, 1, H, W), lambda b: (b, 3, 0, 0)),
        pl.BlockSpec((1, 1, H, W), lambda b: (b, 4, 0, 0)),
    ],
    out_specs=pl.BlockSpec((1, 1, 8), lambda b: (b, 0, 0),
                           memory_space=pltpu.SMEM),
    out_shape=jax.ShapeDtypeStruct((BT, 1, 8), jnp.float32),
)


@jax.jit
def kernel(predictions, heart):
    heart3d = jnp.squeeze(heart, axis=1)
    sc_partials = _sc_call(predictions, heart3d)
    tc_partials = _tc_call(heart, predictions, predictions,
                           predictions, predictions)
    sums = (jnp.sum(sc_partials, axis=(0, 2))
            + jnp.sum(tc_partials[:, 0, :5], axis=0))
    n = sums[4]
    probs = sums[:4] / n
    return jnp.mean((probs - jnp.float32(0.25)) ** 2)
